# Initial kernel scaffold; baseline (speedup 1.0000x reference)
import jax
import jax.numpy as jnp
from jax.experimental import pallas as pl


def kernel(users, items, edge_index, edge_weight, user_weight, item_weight, user_emb0, item_emb0):
    raise NotImplementedError("write your pallas kernel here")


# speedup vs baseline: 3.1686x; 3.1686x over previous
import functools

import jax
import jax.numpy as jnp
from jax import lax
from jax.experimental import pallas as pl
from jax.experimental.pallas import tpu as pltpu
import jax.experimental.pallas.tpu_sc as plsc

NUSERS = 2000
NITEMS = 8000
NNODES = NUSERS + NITEMS
NEDGES = 320000
DIM = 128
NLAYERS = 3
ALPHA_C = 0.1
BATCH = 4096

NC = 2
NS = 16
NW = NC * NS
LANES = 16

NPAD = 10112
CK = 128
EPT = 10112
EPAD = EPT * NW
NCHUNK = EPT // CK
RPT = NPAD // NS


def _scatter_body(table_h, src_h, dst_h, w_h, zeros_h, part_h,
                  accum_s, sidx_s, didx_s, w_s, rows_s, sem):
  c = lax.axis_index("c")
  s = lax.axis_index("s")
  wid = c * NS + s

  pltpu.sync_copy(zeros_h.at[pl.ds(s * RPT, RPT)],
                  accum_s.at[pl.ds(s * RPT, RPT)])
  plsc.subcore_barrier()

  def chunk(i, carry):
    base = wid * EPT + i * CK
    pltpu.sync_copy(src_h.at[pl.ds(base, CK)], sidx_s)
    pltpu.sync_copy(dst_h.at[pl.ds(base, CK)], didx_s)
    pltpu.sync_copy(w_h.at[pl.ds(base, CK)], w_s)
    pltpu.async_copy(table_h.at[sidx_s], rows_s, sem).wait()

    def weight_group(g, carry2):
      wv16 = w_s[pl.ds(g * LANES, LANES)]
      for j2 in range(LANES):
        j = g * LANES + j2
        wsplat = jnp.full((LANES,), wv16[j2], dtype=jnp.float32)
        for cc in range(DIM // LANES):
          sl = pl.ds(cc * LANES, LANES)
          rows_s[j, sl] = rows_s[j, sl] * wsplat
      return carry2

    lax.fori_loop(0, CK // LANES, weight_group, 0)
    pltpu.sync_copy(rows_s, accum_s.at[didx_s], add=True)
    return carry

  lax.fori_loop(0, NCHUNK, chunk, 0)
  plsc.subcore_barrier()
  pltpu.sync_copy(accum_s.at[pl.ds(s * RPT, RPT)],
                  part_h.at[c, pl.ds(s * RPT, RPT)])


def _sc_scatter(table, src, dst, w, zeros):
  mesh = plsc.VectorSubcoreMesh(core_axis_name="c", subcore_axis_name="s")
  return pl.kernel(
      _scatter_body,
      out_type=jax.ShapeDtypeStruct((NC, NPAD, DIM), jnp.float32),
      mesh=mesh,
      scratch_types=[
          pltpu.VMEM_SHARED((NPAD, DIM), jnp.float32),
          pltpu.VMEM((CK,), jnp.int32),
          pltpu.VMEM((CK,), jnp.int32),
          pltpu.VMEM((CK,), jnp.float32),
          pltpu.VMEM((CK, DIM), jnp.float32),
          pltpu.SemaphoreType.DMA,
      ],
  )(table, src, dst, w, zeros)


def _combine_body(coef_ref, p_ref, emb0_ref, acc_ref, table_o, acc_o):
  new = p_ref[0] + p_ref[1] + ALPHA_C * emb0_ref[...]
  table_o[...] = new
  acc_o[...] = acc_ref[...] + coef_ref[0] * new


def _tc_combine(part, emb0p, acc, coef):
  grid = 4
  blk = NPAD // grid
  coef_arr = jnp.full((1,), coef, dtype=jnp.float32)
  return pl.pallas_call(
      _combine_body,
      grid=(grid,),
      in_specs=[
          pl.BlockSpec(memory_space=pltpu.SMEM),
          pl.BlockSpec((NC, blk, DIM), lambda i: (0, i, 0)),
          pl.BlockSpec((blk, DIM), lambda i: (i, 0)),
          pl.BlockSpec((blk, DIM), lambda i: (i, 0)),
      ],
      out_specs=[
          pl.BlockSpec((blk, DIM), lambda i: (i, 0)),
          pl.BlockSpec((blk, DIM), lambda i: (i, 0)),
      ],
      out_shape=[
          jax.ShapeDtypeStruct((NPAD, DIM), jnp.float32),
          jax.ShapeDtypeStruct((NPAD, DIM), jnp.float32),
      ],
  )(coef_arr, part, emb0p, acc)


BPT = BATCH // NW


def _final_body(acc_h, uidx_h, iidx_h, urows_h, irows_h,
                uix_s, iix_s, urows_s, irows_s, sem):
  c = lax.axis_index("c")
  s = lax.axis_index("s")
  wid = c * NS + s
  base = wid * BPT
  pltpu.sync_copy(uidx_h.at[pl.ds(base, BPT)], uix_s)
  pltpu.sync_copy(iidx_h.at[pl.ds(base, BPT)], iix_s)
  pltpu.async_copy(acc_h.at[uix_s], urows_s, sem).wait()
  pltpu.async_copy(acc_h.at[iix_s], irows_s, sem).wait()
  pltpu.sync_copy(urows_s, urows_h.at[pl.ds(base, BPT)])
  pltpu.sync_copy(irows_s, irows_h.at[pl.ds(base, BPT)])


def _sc_final_gather(acc, uidx, iidx):
  mesh = plsc.VectorSubcoreMesh(core_axis_name="c", subcore_axis_name="s")
  return pl.kernel(
      _final_body,
      out_type=[
          jax.ShapeDtypeStruct((BATCH, DIM), jnp.float32),
          jax.ShapeDtypeStruct((BATCH, DIM), jnp.float32),
      ],
      mesh=mesh,
      scratch_types=[
          pltpu.VMEM((BPT,), jnp.int32),
          pltpu.VMEM((BPT,), jnp.int32),
          pltpu.VMEM((BPT, DIM), jnp.float32),
          pltpu.VMEM((BPT, DIM), jnp.float32),
          pltpu.SemaphoreType.DMA,
      ],
  )(acc, uidx, iidx)


def _dot_body(u_ref, i_ref, g_ref):
  g_ref[...] = (jnp.sum(u_ref[...] * i_ref[...], axis=1)
                * jnp.float32(1.0 / 16.0)).reshape(g_ref.shape)


def _tc_dot(urows, irows):
  return pl.pallas_call(
      _dot_body,
      out_shape=jax.ShapeDtypeStruct((BATCH // 128, 128), jnp.float32),
  )(urows, irows)


def kernel(users, items, edge_index, edge_weight, user_weight, item_weight,
           user_emb0, item_emb0):
  f32 = jnp.float32
  table = jnp.concatenate([user_weight, item_weight], axis=0).astype(f32)
  emb0 = jnp.concatenate([user_emb0, item_emb0], axis=0).astype(f32)
  rowpad = jnp.zeros((NPAD - NNODES, DIM), f32)
  table = jnp.concatenate([table, rowpad], axis=0)
  emb0p = jnp.concatenate([emb0, rowpad], axis=0)

  npad_e = EPAD - NEDGES
  dst = jnp.concatenate([edge_index[0].astype(jnp.int32),
                         jnp.full((npad_e,), NNODES, jnp.int32)])
  src = jnp.concatenate([edge_index[1].astype(jnp.int32),
                         jnp.zeros((npad_e,), jnp.int32)])
  w = jnp.concatenate([edge_weight.astype(f32), jnp.zeros((npad_e,), f32)])
  zeros = jnp.zeros((NPAD, DIM), f32)

  acc = table
  for layer in range(NLAYERS):
    part = _sc_scatter(table, src, dst, w, zeros)
    table, acc = _tc_combine(part, emb0p, acc, float(NLAYERS - layer))

  uidx = users.astype(jnp.int32)
  iidx = items.astype(jnp.int32) + NUSERS
  urows, irows = _sc_final_gather(acc, uidx, iidx)
  gamma = _tc_dot(urows, irows).reshape(BATCH)
  return gamma
